# Initial kernel scaffold; baseline (speedup 1.0000x reference)
import jax
import jax.numpy as jnp
from jax.experimental import pallas as pl


def kernel(x, edge_index, y, W0_0, W1_0, b_0, W0_1, W1_1, b_1, W0_2, W1_2, b_2, W0_3, W1_3, b_3):
    raise NotImplementedError("write your pallas kernel here")


# speedup vs baseline: 3.0815x; 3.0815x over previous
import functools

import jax
import jax.numpy as jnp
from jax import lax
from jax.experimental import pallas as pl
from jax.experimental.pallas import tpu as pltpu
from jax.experimental.pallas import tpu_sc as plsc

_N = 10000
_E = 160000
_ACC = 10240
_EPB = 1280
_EP = _EPB * 128


def _mesh():
    return plsc.VectorSubcoreMesh(core_axis_name="c", subcore_axis_name="s")


def _seg_chunked(nch, cps):
    BT = _EPB // 16

    @functools.partial(
        pl.kernel,
        out_type=jax.ShapeDtypeStruct((nch, _ACC, 128), jnp.float32),
        mesh=_mesh(),
        scratch_types=[
            pltpu.VMEM((BT * 128,), jnp.int32),
            pltpu.VMEM((BT, 128), jnp.int32),
            pltpu.VMEM((128, 128), jnp.float32),
            pltpu.VMEM_SHARED((_ACC, 128), jnp.float32),
        ],
    )
    def k(tbl, srcf, dst2, zer, out, gidx, dstb, gbuf, acc):
        sc = lax.axis_index("c")
        t = lax.axis_index("s")
        pltpu.sync_copy(srcf.at[pl.ds(t * (BT * 128), BT * 128)], gidx)
        pltpu.sync_copy(dst2.at[pl.ds(t * BT, BT)], dstb)

        def chunk_body(cc, carry):
            pltpu.sync_copy(zer, gbuf)
            for z in range(_ACC // 16 // 128):
                pltpu.sync_copy(gbuf, acc.at[pl.ds(t * 640 + z * 128, 128)])

            def sbody(i, carry2):
                v = gidx[pl.ds(i * 16, 16)]
                gidx[pl.ds(i * 16, 16)] = jnp.where(
                    cc == 0, v * nch + sc * cps, v + 1)
                return carry2

            lax.fori_loop(0, BT * 8, sbody, 0)
            plsc.subcore_barrier()

            def ebody(b, carry2):
                pltpu.sync_copy(tbl.at[gidx.at[pl.ds(b * 128, 128)]], gbuf)
                pltpu.sync_copy(gbuf, acc.at[dstb.at[b]], add=True)
                return carry2

            lax.fori_loop(0, BT, ebody, 0)
            plsc.subcore_barrier()
            pltpu.sync_copy(acc.at[pl.ds(t * 640, 640)],
                            out.at[sc * cps + cc, pl.ds(t * 640, 640)])
            return carry

        lax.fori_loop(0, cps, chunk_body, 0)

    return k


def _seg_split(row_w):
    BT = _EPB // 32

    @functools.partial(
        pl.kernel,
        out_type=jax.ShapeDtypeStruct((2, _ACC, row_w), jnp.float32),
        mesh=_mesh(),
        scratch_types=[
            pltpu.VMEM((BT * 128,), jnp.int32),
            pltpu.VMEM((BT, 128), jnp.int32),
            pltpu.VMEM((128, row_w), jnp.float32),
            pltpu.VMEM((128, row_w), jnp.float32),
            pltpu.VMEM_SHARED((_ACC, row_w), jnp.float32),
        ],
    )
    def k(tbl, srcf, dst2, zer, out, srcv, dstb, gbuf, zbuf, acc):
        sc = lax.axis_index("c")
        t = lax.axis_index("s")
        w = sc * 16 + t
        pltpu.sync_copy(srcf.at[pl.ds(w * (BT * 128), BT * 128)], srcv)
        pltpu.sync_copy(dst2.at[pl.ds(w * BT, BT)], dstb)
        pltpu.sync_copy(zer, zbuf)
        for z in range(_ACC // 16 // 128):
            pltpu.sync_copy(zbuf, acc.at[pl.ds(t * 640 + z * 128, 128)])
        plsc.subcore_barrier()

        def ebody(b, carry):
            pltpu.sync_copy(tbl.at[srcv.at[pl.ds(b * 128, 128)]], gbuf)
            pltpu.sync_copy(gbuf, acc.at[dstb.at[b]], add=True)
            return carry

        lax.fori_loop(0, BT, ebody, 0)
        plsc.subcore_barrier()
        pltpu.sync_copy(acc.at[pl.ds(t * 640, 640)],
                        out.at[sc, pl.ds(t * 640, 640)])

    return k


def _deg_kernel():
    BT = _EPB // 32

    @functools.partial(
        pl.kernel,
        out_type=jax.ShapeDtypeStruct((2, _ACC, 128), jnp.float32),
        mesh=_mesh(),
        scratch_types=[
            pltpu.VMEM((BT, 128), jnp.int32),
            pltpu.VMEM((128, 128), jnp.float32),
            pltpu.VMEM((128, 128), jnp.float32),
            pltpu.VMEM_SHARED((_ACC, 128), jnp.float32),
        ],
    )
    def k(src2, ones, zer, out, dstb, gbuf, zbuf, acc):
        sc = lax.axis_index("c")
        t = lax.axis_index("s")
        w = sc * 16 + t
        pltpu.sync_copy(src2.at[pl.ds(w * BT, BT)], dstb)
        pltpu.sync_copy(ones, gbuf)
        pltpu.sync_copy(zer, zbuf)
        for z in range(_ACC // 16 // 128):
            pltpu.sync_copy(zbuf, acc.at[pl.ds(t * 640 + z * 128, 128)])
        plsc.subcore_barrier()

        def ebody(b, carry):
            pltpu.sync_copy(gbuf, acc.at[dstb.at[b]], add=True)
            return carry

        lax.fori_loop(0, BT, ebody, 0)
        plsc.subcore_barrier()
        pltpu.sync_copy(acc.at[pl.ds(t * 640, 640)],
                        out.at[sc, pl.ds(t * 640, 640)])

    return k


_deg_k = _deg_kernel()
_seg2_k = _seg_chunked(2, 1)
_seg8_k = _seg_chunked(8, 4)
_seg64_k = _seg_split(128)


def _tc_layer(din, dout, relu, emit_hp):
    nch = din // 128
    BN = 1000
    BJ = min(dout, 512)
    NK = max(1, din // 256)
    grid = (_N // BN, dout // BJ, NK)
    spc = 2 if NK > 1 else nch

    def body(h_ref, S_ref, W0_ref, W1_ref, b_ref, dis_ref, *rest):
        if emit_hp:
            out_ref, hp_ref, acc1, acc2 = rest
        else:
            out_ref, acc1, acc2 = rest
        kk = pl.program_id(2)

        @pl.when(kk == 0)
        def _():
            acc1[...] = jnp.zeros_like(acc1)
            acc2[...] = jnp.zeros_like(acc2)

        acc1[...] += jnp.dot(h_ref[...], W0_ref[...],
                             preferred_element_type=jnp.float32)
        a2 = jnp.dot(S_ref[0], W1_ref[0:128, :],
                     preferred_element_type=jnp.float32)
        for cc in range(1, spc):
            a2 += jnp.dot(S_ref[cc], W1_ref[cc * 128:(cc + 1) * 128, :],
                          preferred_element_type=jnp.float32)
        acc2[...] += a2

        @pl.when(kk == NK - 1)
        def _():
            o = acc1[...] - dis_ref[...] * acc2[...] + b_ref[...]
            if relu:
                o = jnp.maximum(o, 0.0)
            out_ref[...] = o
            if emit_hp:
                hp_ref[...] = dis_ref[...] * o

    BK = spc * 128
    out_shapes = [jax.ShapeDtypeStruct((_N, dout), jnp.float32)]
    if emit_hp:
        out_shapes.append(jax.ShapeDtypeStruct((_N, dout), jnp.float32))
    return pl.pallas_call(
        body,
        grid=grid,
        in_specs=[
            pl.BlockSpec((BN, BK), lambda i, j, k: (i, k)),
            pl.BlockSpec((spc, BN, 128), lambda i, j, k: (k, i, 0)),
            pl.BlockSpec((BK, BJ), lambda i, j, k: (k, j)),
            pl.BlockSpec((BK, BJ), lambda i, j, k: (k, j)),
            pl.BlockSpec((1, BJ), lambda i, j, k: (0, j)),
            pl.BlockSpec((BN, 1), lambda i, j, k: (i, 0)),
        ],
        out_specs=[pl.BlockSpec((BN, BJ), lambda i, j, k: (i, j))
                   for _ in out_shapes],
        out_shape=out_shapes,
        scratch_shapes=[pltpu.VMEM((BN, BJ), jnp.float32),
                        pltpu.VMEM((BN, BJ), jnp.float32)],
        compiler_params=pltpu.CompilerParams(
            dimension_semantics=("parallel", "parallel", "arbitrary")),
    )


def _tc_layer3(din, dout):
    BN = 1000
    NK = din // 256
    grid = (_N // BN, NK)

    def body(h_ref, W0_ref, W1_ref, b_ref, dis_ref, o_ref, g_ref, acc1, acc2):
        kk = pl.program_id(1)

        @pl.when(kk == 0)
        def _():
            acc1[...] = jnp.zeros_like(acc1)
            acc2[...] = jnp.zeros_like(acc2)

        acc1[...] += jnp.dot(h_ref[...], W0_ref[...],
                             preferred_element_type=jnp.float32)
        acc2[...] += jnp.dot(h_ref[...], W1_ref[...],
                             preferred_element_type=jnp.float32)

        @pl.when(kk == NK - 1)
        def _():
            o_ref[...] = acc1[...] + b_ref[...]
            g_ref[...] = jnp.concatenate(
                [dis_ref[...] * acc2[...],
                 jnp.zeros((BN, 128 - dout), jnp.float32)], axis=1)

    return pl.pallas_call(
        body,
        grid=grid,
        in_specs=[
            pl.BlockSpec((BN, 256), lambda i, k: (i, k)),
            pl.BlockSpec((256, dout), lambda i, k: (k, 0)),
            pl.BlockSpec((256, dout), lambda i, k: (k, 0)),
            pl.BlockSpec((1, dout), lambda i, k: (0, 0)),
            pl.BlockSpec((BN, 1), lambda i, k: (i, 0)),
        ],
        out_specs=[pl.BlockSpec((BN, dout), lambda i, k: (i, 0)),
                   pl.BlockSpec((BN, 128), lambda i, k: (i, 0))],
        out_shape=[jax.ShapeDtypeStruct((_N, dout), jnp.float32),
                   jax.ShapeDtypeStruct((_N, 128), jnp.float32)],
        scratch_shapes=[pltpu.VMEM((BN, dout), jnp.float32),
                        pltpu.VMEM((BN, dout), jnp.float32)],
        compiler_params=pltpu.CompilerParams(
            dimension_semantics=("parallel", "arbitrary")),
    )


_tc0 = _tc_layer(256, 1024, True, True)
_tc12 = _tc_layer(1024, 1024, True, True)
_tc2last = _tc_layer(1024, 1024, True, False)
_tc3 = _tc_layer3(1024, 64)


def kernel(x, edge_index, y, W0_0, W1_0, b_0, W0_1, W1_1, b_1,
           W0_2, W1_2, b_2, W0_3, W1_3, b_3):
    f32 = jnp.float32
    i32 = jnp.int32
    src = edge_index[0]
    dst = edge_index[1]
    padn = _EP - _E
    srcg = jnp.concatenate([src, jnp.zeros((padn,), i32)])
    srcN = jnp.concatenate([src, jnp.full((padn,), _N, i32)]).reshape(_EPB, 128)
    dst2 = jnp.concatenate([dst, jnp.full((padn,), _N, i32)]).reshape(_EPB, 128)
    ones128 = jnp.ones((128, 128), f32)
    zer128 = jnp.zeros((128, 128), f32)

    degp = _deg_k(srcN, ones128, zer128)
    deg = degp[0, :_N, 0] + degp[1, :_N, 0]
    dis = jnp.where(deg > 0, lax.rsqrt(deg), 0.0)
    dis1 = dis[:, None]

    hp0 = dis1 * x
    S0 = _seg2_k(hp0.reshape(_N * 2, 128), srcg, dst2, zer128)
    h1, hp1 = _tc0(x, S0, W0_0, W1_0, b_0.reshape(1, -1), dis1)
    S1 = _seg8_k(hp1.reshape(_N * 8, 128), srcg, dst2, zer128)
    h2, hp2 = _tc12(h1, S1, W0_1, W1_1, b_1.reshape(1, -1), dis1)
    S2 = _seg8_k(hp2.reshape(_N * 8, 128), srcg, dst2, zer128)
    h3 = _tc2last(h2, S2, W0_2, W1_2, b_2.reshape(1, -1), dis1)[0]
    o3a, gp = _tc3(h3, W0_3, W1_3, b_3.reshape(1, -1), dis1)
    S3 = _seg64_k(gp, srcg, dst2, zer128)
    return o3a - dis1 * (S3[0, :_N, :64] + S3[1, :_N, :64])


# speedup vs baseline: 3.6033x; 1.1693x over previous
import functools

import jax
import jax.numpy as jnp
from jax import lax
from jax.experimental import pallas as pl
from jax.experimental.pallas import tpu as pltpu
from jax.experimental.pallas import tpu_sc as plsc

_N = 10000
_E = 160000
_ACC = 10240
_EPB = 1280
_EP = _EPB * 128


def _mesh():
    return plsc.VectorSubcoreMesh(core_axis_name="c", subcore_axis_name="s")


def _seg_chunked(nch, cps):
    BT = _EPB // 16

    @functools.partial(
        pl.kernel,
        out_type=jax.ShapeDtypeStruct((nch, _ACC, 128), jnp.float32),
        mesh=_mesh(),
        scratch_types=[
            pltpu.VMEM((BT * 128,), jnp.int32),
            pltpu.VMEM((BT // 2, 128), jnp.int32),
            pltpu.VMEM((128, 128), jnp.float32),
            pltpu.VMEM((128, 128), jnp.float32),
            pltpu.VMEM_SHARED((_ACC, 128), jnp.float32),
            pltpu.SemaphoreType.DMA,
            pltpu.SemaphoreType.DMA,
        ],
    )
    def k(tbl, srcf, dst2, zer, out, gidx, dstb, gA, gB, acc, semA, semB):
        sc = lax.axis_index("c")
        t = lax.axis_index("s")
        HB = BT // 2
        pltpu.sync_copy(srcf.at[pl.ds(t * (BT * 128), BT * 128)], gidx)

        def chunk_body(cc, carry):
            pltpu.sync_copy(zer, gA)
            for z in range(_ACC // 16 // 128):
                pltpu.sync_copy(gA, acc.at[pl.ds(t * 640 + z * 128, 128)])

            def sbody(i, carry2):
                v = gidx[pl.ds(i * 16, 16)]
                gidx[pl.ds(i * 16, 16)] = jnp.where(
                    cc == 0, v * nch + sc * cps, v + 1)
                return carry2

            lax.fori_loop(0, BT * 8, sbody, 0)
            plsc.subcore_barrier()

            for h in range(2):
                off = h * HB
                pltpu.sync_copy(dst2.at[pl.ds(t * BT + off, HB)], dstb)
                pltpu.async_copy(
                    tbl.at[gidx.at[pl.ds(off * 128, 128)]], gA, semA)

                def ebody(p, carry2):
                    b0 = off + 2 * p
                    pltpu.async_copy(
                        tbl.at[gidx.at[pl.ds((b0 + 1) * 128, 128)]],
                        gB, semB)
                    pltpu.make_async_copy(
                        tbl.at[gidx.at[pl.ds(b0 * 128, 128)]],
                        gA, semA).wait()
                    pltpu.sync_copy(gA, acc.at[dstb.at[2 * p]], add=True)

                    @pl.when(p + 1 < HB // 2)
                    def _():
                        pltpu.async_copy(
                            tbl.at[gidx.at[pl.ds((b0 + 2) * 128, 128)]],
                            gA, semA)
                    pltpu.make_async_copy(
                        tbl.at[gidx.at[pl.ds((b0 + 1) * 128, 128)]],
                        gB, semB).wait()
                    pltpu.sync_copy(gB, acc.at[dstb.at[2 * p + 1]], add=True)
                    return carry2

                lax.fori_loop(0, HB // 2, ebody, 0)
            plsc.subcore_barrier()
            pltpu.sync_copy(acc.at[pl.ds(t * 640, 640)],
                            out.at[sc * cps + cc, pl.ds(t * 640, 640)])
            return carry

        lax.fori_loop(0, cps, chunk_body, 0)

    return k


def _seg_split(row_w):
    BT = _EPB // 32

    @functools.partial(
        pl.kernel,
        out_type=jax.ShapeDtypeStruct((2, _ACC, row_w), jnp.float32),
        mesh=_mesh(),
        scratch_types=[
            pltpu.VMEM((BT * 128,), jnp.int32),
            pltpu.VMEM((BT, 128), jnp.int32),
            pltpu.VMEM((128, row_w), jnp.float32),
            pltpu.VMEM((128, row_w), jnp.float32),
            pltpu.VMEM_SHARED((_ACC, row_w), jnp.float32),
        ],
    )
    def k(tbl, srcf, dst2, zer, out, srcv, dstb, gbuf, zbuf, acc):
        sc = lax.axis_index("c")
        t = lax.axis_index("s")
        w = sc * 16 + t
        pltpu.sync_copy(srcf.at[pl.ds(w * (BT * 128), BT * 128)], srcv)
        pltpu.sync_copy(dst2.at[pl.ds(w * BT, BT)], dstb)
        pltpu.sync_copy(zer, zbuf)
        for z in range(_ACC // 16 // 128):
            pltpu.sync_copy(zbuf, acc.at[pl.ds(t * 640 + z * 128, 128)])
        plsc.subcore_barrier()

        def ebody(b, carry):
            pltpu.sync_copy(tbl.at[srcv.at[pl.ds(b * 128, 128)]], gbuf)
            pltpu.sync_copy(gbuf, acc.at[dstb.at[b]], add=True)
            return carry

        lax.fori_loop(0, BT, ebody, 0)
        plsc.subcore_barrier()
        pltpu.sync_copy(acc.at[pl.ds(t * 640, 640)],
                        out.at[sc, pl.ds(t * 640, 640)])

    return k


def _deg_kernel():
    BT = _EPB // 32

    @functools.partial(
        pl.kernel,
        out_type=jax.ShapeDtypeStruct((2, _ACC, 128), jnp.float32),
        mesh=_mesh(),
        scratch_types=[
            pltpu.VMEM((BT, 128), jnp.int32),
            pltpu.VMEM((128, 128), jnp.float32),
            pltpu.VMEM((128, 128), jnp.float32),
            pltpu.VMEM_SHARED((_ACC, 128), jnp.float32),
        ],
    )
    def k(src2, ones, zer, out, dstb, gbuf, zbuf, acc):
        sc = lax.axis_index("c")
        t = lax.axis_index("s")
        w = sc * 16 + t
        pltpu.sync_copy(src2.at[pl.ds(w * BT, BT)], dstb)
        pltpu.sync_copy(ones, gbuf)
        pltpu.sync_copy(zer, zbuf)
        for z in range(_ACC // 16 // 128):
            pltpu.sync_copy(zbuf, acc.at[pl.ds(t * 640 + z * 128, 128)])
        plsc.subcore_barrier()

        def ebody(b, carry):
            pltpu.sync_copy(gbuf, acc.at[dstb.at[b]], add=True)
            return carry

        lax.fori_loop(0, BT, ebody, 0)
        plsc.subcore_barrier()
        pltpu.sync_copy(acc.at[pl.ds(t * 640, 640)],
                        out.at[sc, pl.ds(t * 640, 640)])

    return k


_deg_k = _deg_kernel()
_seg2_k = _seg_chunked(2, 1)
_seg8_k = _seg_chunked(8, 4)
_seg64_k = _seg_split(128)


def _tc_layer(din, dout, relu, emit_hp):
    nch = din // 128
    BN = 1000
    BJ = min(dout, 512)
    NK = max(1, din // 256)
    grid = (_N // BN, dout // BJ, NK)
    spc = 2 if NK > 1 else nch

    def body(h_ref, S_ref, W0_ref, W1_ref, b_ref, dis_ref, *rest):
        if emit_hp:
            out_ref, hp_ref, acc1, acc2 = rest
        else:
            out_ref, acc1, acc2 = rest
        kk = pl.program_id(2)

        @pl.when(kk == 0)
        def _():
            acc1[...] = jnp.zeros_like(acc1)
            acc2[...] = jnp.zeros_like(acc2)

        acc1[...] += jnp.dot(h_ref[...], W0_ref[...],
                             preferred_element_type=jnp.float32)
        a2 = jnp.dot(S_ref[0], W1_ref[0:128, :],
                     preferred_element_type=jnp.float32)
        for cc in range(1, spc):
            a2 += jnp.dot(S_ref[cc], W1_ref[cc * 128:(cc + 1) * 128, :],
                          preferred_element_type=jnp.float32)
        acc2[...] += a2

        @pl.when(kk == NK - 1)
        def _():
            o = acc1[...] - dis_ref[...] * acc2[...] + b_ref[...]
            if relu:
                o = jnp.maximum(o, 0.0)
            out_ref[...] = o
            if emit_hp:
                hp_ref[...] = dis_ref[...] * o

    BK = spc * 128
    out_shapes = [jax.ShapeDtypeStruct((_N, dout), jnp.float32)]
    if emit_hp:
        out_shapes.append(jax.ShapeDtypeStruct((_N, dout), jnp.float32))
    return pl.pallas_call(
        body,
        grid=grid,
        in_specs=[
            pl.BlockSpec((BN, BK), lambda i, j, k: (i, k)),
            pl.BlockSpec((spc, BN, 128), lambda i, j, k: (k, i, 0)),
            pl.BlockSpec((BK, BJ), lambda i, j, k: (k, j)),
            pl.BlockSpec((BK, BJ), lambda i, j, k: (k, j)),
            pl.BlockSpec((1, BJ), lambda i, j, k: (0, j)),
            pl.BlockSpec((BN, 1), lambda i, j, k: (i, 0)),
        ],
        out_specs=[pl.BlockSpec((BN, BJ), lambda i, j, k: (i, j))
                   for _ in out_shapes],
        out_shape=out_shapes,
        scratch_shapes=[pltpu.VMEM((BN, BJ), jnp.float32),
                        pltpu.VMEM((BN, BJ), jnp.float32)],
        compiler_params=pltpu.CompilerParams(
            dimension_semantics=("parallel", "parallel", "arbitrary")),
    )


def _tc_layer3(din, dout):
    BN = 1000
    NK = din // 256
    grid = (_N // BN, NK)

    def body(h_ref, W0_ref, W1_ref, b_ref, dis_ref, o_ref, g_ref, acc1, acc2):
        kk = pl.program_id(1)

        @pl.when(kk == 0)
        def _():
            acc1[...] = jnp.zeros_like(acc1)
            acc2[...] = jnp.zeros_like(acc2)

        acc1[...] += jnp.dot(h_ref[...], W0_ref[...],
                             preferred_element_type=jnp.float32)
        acc2[...] += jnp.dot(h_ref[...], W1_ref[...],
                             preferred_element_type=jnp.float32)

        @pl.when(kk == NK - 1)
        def _():
            o_ref[...] = acc1[...] + b_ref[...]
            g_ref[...] = jnp.concatenate(
                [dis_ref[...] * acc2[...],
                 jnp.zeros((BN, 128 - dout), jnp.float32)], axis=1)

    return pl.pallas_call(
        body,
        grid=grid,
        in_specs=[
            pl.BlockSpec((BN, 256), lambda i, k: (i, k)),
            pl.BlockSpec((256, dout), lambda i, k: (k, 0)),
            pl.BlockSpec((256, dout), lambda i, k: (k, 0)),
            pl.BlockSpec((1, dout), lambda i, k: (0, 0)),
            pl.BlockSpec((BN, 1), lambda i, k: (i, 0)),
        ],
        out_specs=[pl.BlockSpec((BN, dout), lambda i, k: (i, 0)),
                   pl.BlockSpec((BN, 128), lambda i, k: (i, 0))],
        out_shape=[jax.ShapeDtypeStruct((_N, dout), jnp.float32),
                   jax.ShapeDtypeStruct((_N, 128), jnp.float32)],
        scratch_shapes=[pltpu.VMEM((BN, dout), jnp.float32),
                        pltpu.VMEM((BN, dout), jnp.float32)],
        compiler_params=pltpu.CompilerParams(
            dimension_semantics=("parallel", "arbitrary")),
    )


_tc0 = _tc_layer(256, 1024, True, True)
_tc12 = _tc_layer(1024, 1024, True, True)
_tc2last = _tc_layer(1024, 1024, True, False)
_tc3 = _tc_layer3(1024, 64)


def kernel(x, edge_index, y, W0_0, W1_0, b_0, W0_1, W1_1, b_1,
           W0_2, W1_2, b_2, W0_3, W1_3, b_3):
    f32 = jnp.float32
    i32 = jnp.int32
    src = edge_index[0]
    dst = edge_index[1]
    padn = _EP - _E
    srcg = jnp.concatenate([src, jnp.zeros((padn,), i32)])
    srcN = jnp.concatenate([src, jnp.full((padn,), _N, i32)]).reshape(_EPB, 128)
    dst2 = jnp.concatenate([dst, jnp.full((padn,), _N, i32)]).reshape(_EPB, 128)
    ones128 = jnp.ones((128, 128), f32)
    zer128 = jnp.zeros((128, 128), f32)

    degp = _deg_k(srcN, ones128, zer128)
    deg = degp[0, :_N, 0] + degp[1, :_N, 0]
    dis = jnp.where(deg > 0, lax.rsqrt(deg), 0.0)
    dis1 = dis[:, None]

    hp0 = dis1 * x
    S0 = _seg2_k(hp0.reshape(_N * 2, 128), srcg, dst2, zer128)
    h1, hp1 = _tc0(x, S0, W0_0, W1_0, b_0.reshape(1, -1), dis1)
    S1 = _seg8_k(hp1.reshape(_N * 8, 128), srcg, dst2, zer128)
    h2, hp2 = _tc12(h1, S1, W0_1, W1_1, b_1.reshape(1, -1), dis1)
    S2 = _seg8_k(hp2.reshape(_N * 8, 128), srcg, dst2, zer128)
    h3 = _tc2last(h2, S2, W0_2, W1_2, b_2.reshape(1, -1), dis1)[0]
    o3a, gp = _tc3(h3, W0_3, W1_3, b_3.reshape(1, -1), dis1)
    S3 = _seg64_k(gp, srcg, dst2, zer128)
    return o3a - dis1 * (S3[0, :_N, :64] + S3[1, :_N, :64])
